# Initial kernel scaffold; baseline (speedup 1.0000x reference)
import jax
import jax.numpy as jnp
from jax.experimental import pallas as pl


def kernel(node_features, edge_index, edge_attr, params):
    raise NotImplementedError("write your pallas kernel here")


# speedup vs baseline: 1.4341x; 1.4341x over previous
import functools

import jax
import jax.numpy as jnp
from jax import lax
from jax.experimental import pallas as pl
from jax.experimental.pallas import tpu as pltpu
from jax.experimental.pallas import tpu_sc as plsc

N = 64
E = 4096
H = 256
NC = 2
NS = 16
L = 16
NW = NC * NS
EPW = E // NW


def _prologue_body(nf_ref, ew_ref, eb_ref, ea_ref,
                   we1_ref, we2_ref, we3_ref,
                   mb11_ref, mb12_ref, mb13_ref,
                   mb21_ref, mb22_ref, mb23_ref,
                   tgt_ref, ws1_ref, wt1_ref,
                   x0_ref, ab1_ref, c1_ref, c2_ref, c3_ref,
                   db1_ref, db2_ref, db3_ref):
    x0 = jnp.maximum(
        jnp.dot(nf_ref[...], ew_ref[...], preferred_element_type=jnp.float32, precision=lax.Precision.HIGHEST)
        + eb_ref[...], 0.0)
    x0_ref[...] = x0
    ab1_ref[0:N, :] = jnp.dot(x0, ws1_ref[...],
                              preferred_element_type=jnp.float32, precision=lax.Precision.HIGHEST)
    ab1_ref[N:2 * N, :] = jnp.dot(x0, wt1_ref[...],
                                  preferred_element_type=jnp.float32, precision=lax.Precision.HIGHEST)
    ea = ea_ref[...]
    c1_ref[...] = jnp.dot(ea, we1_ref[...],
                          preferred_element_type=jnp.float32, precision=lax.Precision.HIGHEST) + mb11_ref[...]
    c2_ref[...] = jnp.dot(ea, we2_ref[...],
                          preferred_element_type=jnp.float32, precision=lax.Precision.HIGHEST) + mb12_ref[...]
    c3_ref[...] = jnp.dot(ea, we3_ref[...],
                          preferred_element_type=jnp.float32, precision=lax.Precision.HIGHEST) + mb13_ref[...]
    onehot = (tgt_ref[...] ==
              lax.broadcasted_iota(jnp.int32, (E, N), 1)).astype(jnp.float32)
    deg = jnp.sum(onehot, axis=0)[:, None]
    db1_ref[...] = deg * mb21_ref[...]
    db2_ref[...] = deg * mb22_ref[...]
    db3_ref[...] = deg * mb23_ref[...]


_prologue = pl.pallas_call(
    _prologue_body,
    out_shape=[
        jax.ShapeDtypeStruct((N, H), jnp.float32),
        jax.ShapeDtypeStruct((2 * N, H), jnp.float32),
        jax.ShapeDtypeStruct((E, H), jnp.float32),
        jax.ShapeDtypeStruct((E, H), jnp.float32),
        jax.ShapeDtypeStruct((E, H), jnp.float32),
        jax.ShapeDtypeStruct((N, H), jnp.float32),
        jax.ShapeDtypeStruct((N, H), jnp.float32),
        jax.ShapeDtypeStruct((N, H), jnp.float32),
    ],
)


def _sc_conv_body(ab_hbm, c_hbm, src_hbm, tgt_hbm, out_hbm,
                  ab_v, c_v, src_v, tgt_v, s_v):
    cid = lax.axis_index("c")
    sid = lax.axis_index("s")
    wid = sid * NC + cid
    base = wid * EPW

    pltpu.sync_copy(ab_hbm, ab_v)
    pltpu.sync_copy(c_hbm.at[pl.ds(base, EPW)], c_v)
    pltpu.sync_copy(src_hbm.at[pl.ds(base, EPW)], src_v)
    pltpu.sync_copy(tgt_hbm.at[pl.ds(base, EPW)], tgt_v)

    iota = lax.iota(jnp.int32, 16)
    zf = jnp.zeros((16,), jnp.float32)
    for r in range(N):
        for j in range(H // L):
            s_v[r, 16 * j:16 * j + 16] = zf

    def edge_body(e, carry):
        ev = jnp.full((16,), e, jnp.int32)
        s = plsc.load_gather(src_v, [ev])
        t = plsc.load_gather(tgt_v, [ev])
        tb = t + N
        for j in range(H // L):
            col = iota + 16 * j
            a = plsc.load_gather(ab_v, [s, col])
            b = plsc.load_gather(ab_v, [tb, col])
            cc = plsc.load_gather(c_v, [ev, col])
            h = jnp.maximum(a + b + cc, 0.0)
            plsc.addupdate_scatter(s_v, [t, col], h)
        return carry

    lax.fori_loop(0, EPW, edge_body, 0)

    pltpu.sync_copy(s_v, out_hbm.at[wid])


_sc_conv = functools.partial(
    pl.kernel,
    out_type=jax.ShapeDtypeStruct((NW, N, H), jnp.float32),
    mesh=plsc.VectorSubcoreMesh(core_axis_name="c", subcore_axis_name="s",
                                num_cores=NC, num_subcores=NS),
    compiler_params=pltpu.CompilerParams(needs_layout_passes=False),
    scratch_types=[
        pltpu.VMEM((2 * N, H), jnp.float32),
        pltpu.VMEM((EPW, H), jnp.float32),
        pltpu.VMEM((EPW,), jnp.int32),
        pltpu.VMEM((EPW,), jnp.int32),
        pltpu.VMEM((N, H), jnp.float32),
    ],
)(_sc_conv_body)


def _update_body(x_ref, s2_ref, db_ref, mw2_ref, uw1a_ref, uw1b_ref,
                 ub1_ref, uw2_ref, ub2_ref, wsn_ref, wtn_ref,
                 xn_ref, abn_ref):
    s = jnp.sum(s2_ref[...], axis=0)
    agg = jnp.dot(s, mw2_ref[...], preferred_element_type=jnp.float32, precision=lax.Precision.HIGHEST) \
        + db_ref[...]
    u = jnp.maximum(
        jnp.dot(x_ref[...], uw1a_ref[...], preferred_element_type=jnp.float32, precision=lax.Precision.HIGHEST)
        + jnp.dot(agg, uw1b_ref[...], preferred_element_type=jnp.float32, precision=lax.Precision.HIGHEST)
        + ub1_ref[...], 0.0)
    xn = jnp.maximum(
        jnp.dot(u, uw2_ref[...], preferred_element_type=jnp.float32, precision=lax.Precision.HIGHEST)
        + ub2_ref[...], 0.0)
    xn_ref[...] = xn
    abn_ref[0:N, :] = jnp.dot(xn, wsn_ref[...],
                              preferred_element_type=jnp.float32, precision=lax.Precision.HIGHEST)
    abn_ref[N:2 * N, :] = jnp.dot(xn, wtn_ref[...],
                                  preferred_element_type=jnp.float32, precision=lax.Precision.HIGHEST)


_update = pl.pallas_call(
    _update_body,
    out_shape=[
        jax.ShapeDtypeStruct((N, H), jnp.float32),
        jax.ShapeDtypeStruct((2 * N, H), jnp.float32),
    ],
)


def _update_last_body(x_ref, s2_ref, db_ref, mw2_ref, uw1a_ref, uw1b_ref,
                      ub1_ref, uw2_ref, ub2_ref, xn_ref):
    s = jnp.sum(s2_ref[...], axis=0)
    agg = jnp.dot(s, mw2_ref[...], preferred_element_type=jnp.float32, precision=lax.Precision.HIGHEST) \
        + db_ref[...]
    u = jnp.maximum(
        jnp.dot(x_ref[...], uw1a_ref[...], preferred_element_type=jnp.float32, precision=lax.Precision.HIGHEST)
        + jnp.dot(agg, uw1b_ref[...], preferred_element_type=jnp.float32, precision=lax.Precision.HIGHEST)
        + ub1_ref[...], 0.0)
    xn_ref[...] = jnp.maximum(
        jnp.dot(u, uw2_ref[...], preferred_element_type=jnp.float32, precision=lax.Precision.HIGHEST)
        + ub2_ref[...], 0.0)


_update_last = pl.pallas_call(
    _update_last_body,
    out_shape=[jax.ShapeDtypeStruct((N, H), jnp.float32)],
)


_KB = 2048
_KSTEPS = (N * H) // _KB


def _head1_body(g_ref, pw1_ref, vw1_ref, pb1_ref, vb1_ref, vw2_ref, vb2_ref,
                ph_ref, val_ref, ph_acc, vh_acc):
    k = pl.program_id(0)

    @pl.when(k == 0)
    def _():
        ph_acc[...] = jnp.zeros_like(ph_acc)
        vh_acc[...] = jnp.zeros_like(vh_acc)

    g = g_ref[...]
    ph_acc[...] += jnp.dot(g, pw1_ref[...], preferred_element_type=jnp.float32, precision=lax.Precision.HIGHEST)
    vh_acc[...] += jnp.dot(g, vw1_ref[...], preferred_element_type=jnp.float32, precision=lax.Precision.HIGHEST)

    @pl.when(k == _KSTEPS - 1)
    def _():
        ph_ref[...] = jnp.maximum(ph_acc[...] + pb1_ref[...], 0.0)
        vh = jnp.maximum(vh_acc[...] + vb1_ref[...], 0.0)
        val_ref[...] = jnp.tanh(
            jnp.sum(vh * vw2_ref[...], axis=1, keepdims=True) + vb2_ref[...])


_head1 = pl.pallas_call(
    _head1_body,
    grid=(_KSTEPS,),
    in_specs=[
        pl.BlockSpec((1, _KB), lambda k: (0, k)),
        pl.BlockSpec((_KB, 1024), lambda k: (k, 0)),
        pl.BlockSpec((_KB, 256), lambda k: (k, 0)),
        pl.BlockSpec((1, 1024), lambda k: (0, 0)),
        pl.BlockSpec((1, 256), lambda k: (0, 0)),
        pl.BlockSpec((1, 256), lambda k: (0, 0)),
        pl.BlockSpec((1, 1), lambda k: (0, 0)),
    ],
    out_specs=[
        pl.BlockSpec((1, 1024), lambda k: (0, 0)),
        pl.BlockSpec((1, 1), lambda k: (0, 0)),
    ],
    out_shape=[
        jax.ShapeDtypeStruct((1, 1024), jnp.float32),
        jax.ShapeDtypeStruct((1, 1), jnp.float32),
    ],
    scratch_shapes=[
        pltpu.VMEM((1, 1024), jnp.float32),
        pltpu.VMEM((1, 256), jnp.float32),
    ],
)

_NB = 2560
_NSTEPS = 20480 // _NB


def _head2_body(ph_ref, pw2_ref, pb2_ref, pol_ref):
    pol_ref[...] = jnp.dot(ph_ref[...], pw2_ref[...],
                           preferred_element_type=jnp.float32, precision=lax.Precision.HIGHEST) + pb2_ref[...]


_head2 = pl.pallas_call(
    _head2_body,
    grid=(_NSTEPS,),
    in_specs=[
        pl.BlockSpec((1, 1024), lambda n: (0, 0)),
        pl.BlockSpec((1024, _NB), lambda n: (0, n)),
        pl.BlockSpec((1, _NB), lambda n: (0, n)),
    ],
    out_specs=[pl.BlockSpec((1, _NB), lambda n: (0, n))],
    out_shape=[jax.ShapeDtypeStruct((1, 20480), jnp.float32)],
)


def kernel(node_features, edge_index, edge_attr, params):
    p = params
    src = edge_index[0]
    tgt = edge_index[1]
    convs = (p['conv1'], p['conv2'], p['conv3'])

    nfp = jnp.pad(node_features, ((0, 0), (0, 3)))
    ewp = jnp.pad(p['embed_w'], ((0, 3), (0, 0)))
    eap = jnp.pad(edge_attr, ((0, 0), (0, 4)))
    wes = [jnp.pad(c['mw1'][2 * H:], ((0, 4), (0, 0))) for c in convs]

    x0, ab1, c1, c2, c3, db1, db2, db3 = _prologue(
        nfp, ewp, p['embed_b'][None, :], eap,
        wes[0], wes[1], wes[2],
        convs[0]['mb1'][None, :], convs[1]['mb1'][None, :],
        convs[2]['mb1'][None, :],
        convs[0]['mb2'][None, :], convs[1]['mb2'][None, :],
        convs[2]['mb2'][None, :],
        tgt[:, None], convs[0]['mw1'][:H], convs[0]['mw1'][H:2 * H])

    x, ab = x0, ab1
    cs = (c1, c2, c3)
    dbs = (db1, db2, db3)
    for i in range(3):
        c = convs[i]
        s2 = _sc_conv(ab, cs[i], src, tgt)
        if i < 2:
            nxt = convs[i + 1]
            x, ab = _update(x, s2, dbs[i], c['mw2'], c['uw1'][:H],
                            c['uw1'][H:], c['ub1'][None, :], c['uw2'],
                            c['ub2'][None, :], nxt['mw1'][:H],
                            nxt['mw1'][H:2 * H])
        else:
            (x,) = _update_last(x, s2, dbs[i], c['mw2'], c['uw1'][:H],
                                c['uw1'][H:], c['ub1'][None, :], c['uw2'],
                                c['ub2'][None, :])

    g = x.reshape(1, N * H)
    ph, val = _head1(g, p['pw1'], p['vw1'], p['pb1'][None, :],
                     p['vb1'][None, :], p['vw2'].reshape(1, H),
                     p['vb2'][None, :])
    (pol,) = _head2(ph, p['pw2'], p['pb2'][None, :])
    return pol.reshape(20480), val.reshape(1)


# speedup vs baseline: 1.6437x; 1.1462x over previous
import functools

import jax
import jax.numpy as jnp
from jax import lax
from jax.experimental import pallas as pl
from jax.experimental.pallas import tpu as pltpu
from jax.experimental.pallas import tpu_sc as plsc

N = 64
E = 4096
H = 256
NC = 2
NS = 16
L = 16
NW = NC * NS
EPW = E // NW


def _prologue_body(nf_ref, ew_ref, eb_ref, ea_ref,
                   we1_ref, we2_ref, we3_ref,
                   mb11_ref, mb12_ref, mb13_ref,
                   mb21_ref, mb22_ref, mb23_ref,
                   tgt_ref, ws1_ref, wt1_ref,
                   x0_ref, ab1_ref, c1_ref, c2_ref, c3_ref,
                   db1_ref, db2_ref, db3_ref):
    x0 = jnp.maximum(
        jnp.dot(nf_ref[...], ew_ref[...], preferred_element_type=jnp.float32, precision=lax.Precision.HIGHEST)
        + eb_ref[...], 0.0)
    x0_ref[...] = x0
    ab1_ref[0:N, :] = jnp.dot(x0, ws1_ref[...],
                              preferred_element_type=jnp.float32, precision=lax.Precision.HIGHEST)
    ab1_ref[N:2 * N, :] = jnp.dot(x0, wt1_ref[...],
                                  preferred_element_type=jnp.float32, precision=lax.Precision.HIGHEST)
    ea = ea_ref[...]
    c1_ref[...] = jnp.dot(ea, we1_ref[...],
                          preferred_element_type=jnp.float32, precision=lax.Precision.HIGHEST) + mb11_ref[...]
    c2_ref[...] = jnp.dot(ea, we2_ref[...],
                          preferred_element_type=jnp.float32, precision=lax.Precision.HIGHEST) + mb12_ref[...]
    c3_ref[...] = jnp.dot(ea, we3_ref[...],
                          preferred_element_type=jnp.float32, precision=lax.Precision.HIGHEST) + mb13_ref[...]
    onehot = (tgt_ref[...] ==
              lax.broadcasted_iota(jnp.int32, (E, N), 1)).astype(jnp.float32)
    deg = jnp.sum(onehot, axis=0)[:, None]
    db1_ref[...] = deg * mb21_ref[...]
    db2_ref[...] = deg * mb22_ref[...]
    db3_ref[...] = deg * mb23_ref[...]


_prologue = pl.pallas_call(
    _prologue_body,
    out_shape=[
        jax.ShapeDtypeStruct((N, H), jnp.float32),
        jax.ShapeDtypeStruct((2 * N, H), jnp.float32),
        jax.ShapeDtypeStruct((E, H), jnp.float32),
        jax.ShapeDtypeStruct((E, H), jnp.float32),
        jax.ShapeDtypeStruct((E, H), jnp.float32),
        jax.ShapeDtypeStruct((N, H), jnp.float32),
        jax.ShapeDtypeStruct((N, H), jnp.float32),
        jax.ShapeDtypeStruct((N, H), jnp.float32),
    ],
)


def _sc_conv_body(ab_hbm, c_hbm, src_hbm, tgt_hbm, out_hbm,
                  ab_v, c_v, src_v, tgt_v, s_v):
    cid = lax.axis_index("c")
    sid = lax.axis_index("s")
    wid = sid * NC + cid
    base = wid * EPW

    pltpu.sync_copy(ab_hbm, ab_v)
    pltpu.sync_copy(c_hbm.at[pl.ds(base, EPW)], c_v)
    pltpu.sync_copy(src_hbm.at[pl.ds(base, EPW)], src_v)
    pltpu.sync_copy(tgt_hbm.at[pl.ds(base, EPW)], tgt_v)

    iota = lax.iota(jnp.int32, 16)
    zf = jnp.zeros((16,), jnp.float32)
    for r in range(N):
        for j in range(H // L):
            s_v[r, 16 * j:16 * j + 16] = zf

    @plsc.parallel_loop(0, EPW, step=1, unroll=4)
    def _edges(e):
        ev = jnp.full((16,), e, jnp.int32)
        s = plsc.load_gather(src_v, [ev])
        t = plsc.load_gather(tgt_v, [ev])
        tb = t + N
        for j in range(H // L):
            col = iota + 16 * j
            a = plsc.load_gather(ab_v, [s, col])
            b = plsc.load_gather(ab_v, [tb, col])
            cc = plsc.load_gather(c_v, [ev, col])
            h = jnp.maximum(a + b + cc, 0.0)
            plsc.addupdate_scatter(s_v, [t, col], h)

    pltpu.sync_copy(s_v, out_hbm.at[wid])


_sc_conv = functools.partial(
    pl.kernel,
    out_type=jax.ShapeDtypeStruct((NW, N, H), jnp.float32),
    mesh=plsc.VectorSubcoreMesh(core_axis_name="c", subcore_axis_name="s",
                                num_cores=NC, num_subcores=NS),
    compiler_params=pltpu.CompilerParams(needs_layout_passes=False),
    scratch_types=[
        pltpu.VMEM((2 * N, H), jnp.float32),
        pltpu.VMEM((EPW, H), jnp.float32),
        pltpu.VMEM((EPW,), jnp.int32),
        pltpu.VMEM((EPW,), jnp.int32),
        pltpu.VMEM((N, H), jnp.float32),
    ],
)(_sc_conv_body)


def _update_body(x_ref, s2_ref, db_ref, mw2_ref, uw1a_ref, uw1b_ref,
                 ub1_ref, uw2_ref, ub2_ref, wsn_ref, wtn_ref,
                 xn_ref, abn_ref):
    s = jnp.sum(s2_ref[...], axis=0)
    agg = jnp.dot(s, mw2_ref[...], preferred_element_type=jnp.float32, precision=lax.Precision.HIGHEST) \
        + db_ref[...]
    u = jnp.maximum(
        jnp.dot(x_ref[...], uw1a_ref[...], preferred_element_type=jnp.float32, precision=lax.Precision.HIGHEST)
        + jnp.dot(agg, uw1b_ref[...], preferred_element_type=jnp.float32, precision=lax.Precision.HIGHEST)
        + ub1_ref[...], 0.0)
    xn = jnp.maximum(
        jnp.dot(u, uw2_ref[...], preferred_element_type=jnp.float32, precision=lax.Precision.HIGHEST)
        + ub2_ref[...], 0.0)
    xn_ref[...] = xn
    abn_ref[0:N, :] = jnp.dot(xn, wsn_ref[...],
                              preferred_element_type=jnp.float32, precision=lax.Precision.HIGHEST)
    abn_ref[N:2 * N, :] = jnp.dot(xn, wtn_ref[...],
                                  preferred_element_type=jnp.float32, precision=lax.Precision.HIGHEST)


_update = pl.pallas_call(
    _update_body,
    out_shape=[
        jax.ShapeDtypeStruct((N, H), jnp.float32),
        jax.ShapeDtypeStruct((2 * N, H), jnp.float32),
    ],
)


def _update_last_body(x_ref, s2_ref, db_ref, mw2_ref, uw1a_ref, uw1b_ref,
                      ub1_ref, uw2_ref, ub2_ref, xn_ref):
    s = jnp.sum(s2_ref[...], axis=0)
    agg = jnp.dot(s, mw2_ref[...], preferred_element_type=jnp.float32, precision=lax.Precision.HIGHEST) \
        + db_ref[...]
    u = jnp.maximum(
        jnp.dot(x_ref[...], uw1a_ref[...], preferred_element_type=jnp.float32, precision=lax.Precision.HIGHEST)
        + jnp.dot(agg, uw1b_ref[...], preferred_element_type=jnp.float32, precision=lax.Precision.HIGHEST)
        + ub1_ref[...], 0.0)
    xn_ref[...] = jnp.maximum(
        jnp.dot(u, uw2_ref[...], preferred_element_type=jnp.float32, precision=lax.Precision.HIGHEST)
        + ub2_ref[...], 0.0)


_update_last = pl.pallas_call(
    _update_last_body,
    out_shape=[jax.ShapeDtypeStruct((N, H), jnp.float32)],
)


_KB = 2048
_KSTEPS = (N * H) // _KB


def _head1_body(g_ref, pw1_ref, vw1_ref, pb1_ref, vb1_ref, vw2_ref, vb2_ref,
                ph_ref, val_ref, ph_acc, vh_acc):
    k = pl.program_id(0)

    @pl.when(k == 0)
    def _():
        ph_acc[...] = jnp.zeros_like(ph_acc)
        vh_acc[...] = jnp.zeros_like(vh_acc)

    g = g_ref[...]
    ph_acc[...] += jnp.dot(g, pw1_ref[...], preferred_element_type=jnp.float32, precision=lax.Precision.HIGHEST)
    vh_acc[...] += jnp.dot(g, vw1_ref[...], preferred_element_type=jnp.float32, precision=lax.Precision.HIGHEST)

    @pl.when(k == _KSTEPS - 1)
    def _():
        ph_ref[...] = jnp.maximum(ph_acc[...] + pb1_ref[...], 0.0)
        vh = jnp.maximum(vh_acc[...] + vb1_ref[...], 0.0)
        val_ref[...] = jnp.tanh(
            jnp.sum(vh * vw2_ref[...], axis=1, keepdims=True) + vb2_ref[...])


_head1 = pl.pallas_call(
    _head1_body,
    grid=(_KSTEPS,),
    in_specs=[
        pl.BlockSpec((1, _KB), lambda k: (0, k)),
        pl.BlockSpec((_KB, 1024), lambda k: (k, 0)),
        pl.BlockSpec((_KB, 256), lambda k: (k, 0)),
        pl.BlockSpec((1, 1024), lambda k: (0, 0)),
        pl.BlockSpec((1, 256), lambda k: (0, 0)),
        pl.BlockSpec((1, 256), lambda k: (0, 0)),
        pl.BlockSpec((1, 1), lambda k: (0, 0)),
    ],
    out_specs=[
        pl.BlockSpec((1, 1024), lambda k: (0, 0)),
        pl.BlockSpec((1, 1), lambda k: (0, 0)),
    ],
    out_shape=[
        jax.ShapeDtypeStruct((1, 1024), jnp.float32),
        jax.ShapeDtypeStruct((1, 1), jnp.float32),
    ],
    scratch_shapes=[
        pltpu.VMEM((1, 1024), jnp.float32),
        pltpu.VMEM((1, 256), jnp.float32),
    ],
)

_NB = 2560
_NSTEPS = 20480 // _NB


def _head2_body(ph_ref, pw2_ref, pb2_ref, pol_ref):
    pol_ref[...] = jnp.dot(ph_ref[...], pw2_ref[...],
                           preferred_element_type=jnp.float32, precision=lax.Precision.HIGHEST) + pb2_ref[...]


_head2 = pl.pallas_call(
    _head2_body,
    grid=(_NSTEPS,),
    in_specs=[
        pl.BlockSpec((1, 1024), lambda n: (0, 0)),
        pl.BlockSpec((1024, _NB), lambda n: (0, n)),
        pl.BlockSpec((1, _NB), lambda n: (0, n)),
    ],
    out_specs=[pl.BlockSpec((1, _NB), lambda n: (0, n))],
    out_shape=[jax.ShapeDtypeStruct((1, 20480), jnp.float32)],
)


def kernel(node_features, edge_index, edge_attr, params):
    p = params
    src = edge_index[0]
    tgt = edge_index[1]
    convs = (p['conv1'], p['conv2'], p['conv3'])

    nfp = jnp.pad(node_features, ((0, 0), (0, 3)))
    ewp = jnp.pad(p['embed_w'], ((0, 3), (0, 0)))
    eap = jnp.pad(edge_attr, ((0, 0), (0, 4)))
    wes = [jnp.pad(c['mw1'][2 * H:], ((0, 4), (0, 0))) for c in convs]

    x0, ab1, c1, c2, c3, db1, db2, db3 = _prologue(
        nfp, ewp, p['embed_b'][None, :], eap,
        wes[0], wes[1], wes[2],
        convs[0]['mb1'][None, :], convs[1]['mb1'][None, :],
        convs[2]['mb1'][None, :],
        convs[0]['mb2'][None, :], convs[1]['mb2'][None, :],
        convs[2]['mb2'][None, :],
        tgt[:, None], convs[0]['mw1'][:H], convs[0]['mw1'][H:2 * H])

    x, ab = x0, ab1
    cs = (c1, c2, c3)
    dbs = (db1, db2, db3)
    for i in range(3):
        c = convs[i]
        s2 = _sc_conv(ab, cs[i], src, tgt)
        if i < 2:
            nxt = convs[i + 1]
            x, ab = _update(x, s2, dbs[i], c['mw2'], c['uw1'][:H],
                            c['uw1'][H:], c['ub1'][None, :], c['uw2'],
                            c['ub2'][None, :], nxt['mw1'][:H],
                            nxt['mw1'][H:2 * H])
        else:
            (x,) = _update_last(x, s2, dbs[i], c['mw2'], c['uw1'][:H],
                                c['uw1'][H:], c['ub1'][None, :], c['uw2'],
                                c['ub2'][None, :])

    g = x.reshape(1, N * H)
    ph, val = _head1(g, p['pw1'], p['vw1'], p['pb1'][None, :],
                     p['vb1'][None, :], p['vw2'].reshape(1, H),
                     p['vb2'][None, :])
    (pol,) = _head2(ph, p['pw2'], p['pb2'][None, :])
    return pol.reshape(20480), val.reshape(1)


# speedup vs baseline: 2.1107x; 1.2841x over previous
import functools

import jax
import jax.numpy as jnp
from jax import lax
from jax.experimental import pallas as pl
from jax.experimental.pallas import tpu as pltpu
from jax.experimental.pallas import tpu_sc as plsc

N = 64
E = 4096
H = 256
NC = 2
NS = 16
L = 16
NW = NC * NS
EPW = E // NW

_F32 = jnp.float32


def _mm(a, b):
    return jnp.dot(a, b, preferred_element_type=_F32)


def _prologue_body(nf_ref, ew_ref, eb_ref, ea_ref, mw1_ref, mb1_ref,
                   x0_ref, ab1_ref, c1_ref):
    x0 = jnp.maximum(_mm(nf_ref[...], ew_ref[...]) + eb_ref[...], 0.0)
    x0_ref[...] = x0
    ab1_ref[0:N, :] = _mm(x0, mw1_ref[0:H, :])
    ab1_ref[N:2 * N, :] = _mm(x0, mw1_ref[H:2 * H, :])
    c1_ref[...] = _mm(ea_ref[...], mw1_ref[2 * H:, :]) + mb1_ref[...]


_prologue = pl.pallas_call(
    _prologue_body,
    out_shape=[
        jax.ShapeDtypeStruct((N, H), _F32),
        jax.ShapeDtypeStruct((2 * N, H), _F32),
        jax.ShapeDtypeStruct((E, H), _F32),
    ],
)


def _prologue2_body(ea_ref, mw12_ref, mw13_ref, mb12_ref, mb13_ref, tgt_ref,
                    c2_ref, c3_ref, oht_ref):
    ea = ea_ref[...]
    c2_ref[...] = _mm(ea, mw12_ref[2 * H:, :]) + mb12_ref[...]
    c3_ref[...] = _mm(ea, mw13_ref[2 * H:, :]) + mb13_ref[...]
    oht_ref[...] = (tgt_ref[...] ==
                    lax.broadcasted_iota(jnp.int32, (N, E), 0)).astype(_F32)


_prologue2 = pl.pallas_call(
    _prologue2_body,
    out_shape=[
        jax.ShapeDtypeStruct((E, H), _F32),
        jax.ShapeDtypeStruct((E, H), _F32),
        jax.ShapeDtypeStruct((N, E), _F32),
    ],
)


def _sc_conv_body(ab_hbm, c_hbm, src_hbm, tgt_hbm, out_hbm,
                  ab_v, c_v, src_v, tgt_v, h_v):
    cid = lax.axis_index("c")
    sid = lax.axis_index("s")
    wid = sid * NC + cid
    base = wid * EPW

    pltpu.sync_copy(ab_hbm, ab_v)
    pltpu.sync_copy(c_hbm.at[pl.ds(base, EPW)], c_v)
    pltpu.sync_copy(src_hbm.at[pl.ds(base, EPW)], src_v)
    pltpu.sync_copy(tgt_hbm.at[pl.ds(base, EPW)], tgt_v)

    iota = lax.iota(jnp.int32, 16)

    @plsc.parallel_loop(0, EPW, step=1, unroll=4)
    def _edges(e):
        ev = jnp.full((16,), e, jnp.int32)
        s = plsc.load_gather(src_v, [ev])
        t = plsc.load_gather(tgt_v, [ev]) + N
        for j in range(H // L):
            col = iota + 16 * j
            a = plsc.load_gather(ab_v, [s, col])
            b = plsc.load_gather(ab_v, [t, col])
            cc = plsc.load_gather(c_v, [ev, col])
            h = jnp.maximum(a + b + cc, 0.0)
            plsc.store_scatter(h_v, [ev, col], h)

    pltpu.sync_copy(h_v, out_hbm.at[pl.ds(base, EPW)])


_sc_conv = functools.partial(
    pl.kernel,
    out_type=jax.ShapeDtypeStruct((E, H), _F32),
    mesh=plsc.VectorSubcoreMesh(core_axis_name="c", subcore_axis_name="s",
                                num_cores=NC, num_subcores=NS),
    compiler_params=pltpu.CompilerParams(needs_layout_passes=False),
    scratch_types=[
        pltpu.VMEM((2 * N, H), _F32),
        pltpu.VMEM((EPW, H), _F32),
        pltpu.VMEM((EPW,), jnp.int32),
        pltpu.VMEM((EPW,), jnp.int32),
        pltpu.VMEM((EPW, H), _F32),
    ],
)(_sc_conv_body)


def _agg(m1_ref, oht_ref, mw2_ref, mb2_ref):
    m = _mm(m1_ref[...], mw2_ref[...]) + mb2_ref[...]
    return jnp.dot(oht_ref[...], m, preferred_element_type=_F32,
                   precision=lax.Precision.HIGHEST)


def _update_body(x_ref, m1_ref, oht_ref, mw2_ref, mb2_ref, uw1_ref,
                 ub1_ref, uw2_ref, ub2_ref, mw1n_ref,
                 xn_ref, abn_ref):
    agg = _agg(m1_ref, oht_ref, mw2_ref, mb2_ref)
    u = jnp.maximum(_mm(x_ref[...], uw1_ref[0:H, :])
                    + _mm(agg, uw1_ref[H:, :]) + ub1_ref[...], 0.0)
    xn = jnp.maximum(_mm(u, uw2_ref[...]) + ub2_ref[...], 0.0)
    xn_ref[...] = xn
    abn_ref[0:N, :] = _mm(xn, mw1n_ref[0:H, :])
    abn_ref[N:2 * N, :] = _mm(xn, mw1n_ref[H:2 * H, :])


_update = pl.pallas_call(
    _update_body,
    out_shape=[
        jax.ShapeDtypeStruct((N, H), _F32),
        jax.ShapeDtypeStruct((2 * N, H), _F32),
    ],
)


def _update_last_body(x_ref, m1_ref, oht_ref, mw2_ref, mb2_ref, uw1_ref,
                      ub1_ref, uw2_ref, ub2_ref, g_ref):
    agg = _agg(m1_ref, oht_ref, mw2_ref, mb2_ref)
    u = jnp.maximum(_mm(x_ref[...], uw1_ref[0:H, :])
                    + _mm(agg, uw1_ref[H:, :]) + ub1_ref[...], 0.0)
    g_ref[...] = jnp.maximum(_mm(u, uw2_ref[...]) + ub2_ref[...], 0.0)


_update_last = pl.pallas_call(
    _update_last_body,
    out_shape=[jax.ShapeDtypeStruct((N, H), _F32)],
)


_KB = 2048
_KSTEPS = (N * H) // _KB


def _head1_body(g_ref, pw1_ref, vw1_ref, pb1_ref, vb1_ref, vw2_ref, vb2_ref,
                ph_ref, val_ref, ph_acc, vh_acc):
    k = pl.program_id(0)

    @pl.when(k == 0)
    def _():
        ph_acc[...] = jnp.zeros_like(ph_acc)
        vh_acc[...] = jnp.zeros_like(vh_acc)

    g = g_ref[...]
    ph_acc[...] += _mm(g, pw1_ref[...])
    vh_acc[...] += _mm(g, vw1_ref[...])

    @pl.when(k == _KSTEPS - 1)
    def _():
        ph_ref[...] = jnp.maximum(ph_acc[...] + pb1_ref[...], 0.0)
        vh = jnp.maximum(vh_acc[...] + vb1_ref[...], 0.0)
        val_ref[...] = jnp.tanh(
            jnp.sum(vh * vw2_ref[...], axis=1, keepdims=True) + vb2_ref[...])


_head1 = pl.pallas_call(
    _head1_body,
    grid=(_KSTEPS,),
    in_specs=[
        pl.BlockSpec((1, _KB), lambda k: (0, k)),
        pl.BlockSpec((_KB, 1024), lambda k: (k, 0)),
        pl.BlockSpec((_KB, 256), lambda k: (k, 0)),
        pl.BlockSpec((1, 1024), lambda k: (0, 0)),
        pl.BlockSpec((1, 256), lambda k: (0, 0)),
        pl.BlockSpec((1, 256), lambda k: (0, 0)),
        pl.BlockSpec((1, 1), lambda k: (0, 0)),
    ],
    out_specs=[
        pl.BlockSpec((1, 1024), lambda k: (0, 0)),
        pl.BlockSpec((1, 1), lambda k: (0, 0)),
    ],
    out_shape=[
        jax.ShapeDtypeStruct((1, 1024), _F32),
        jax.ShapeDtypeStruct((1, 1), _F32),
    ],
    scratch_shapes=[
        pltpu.VMEM((1, 1024), _F32),
        pltpu.VMEM((1, 256), _F32),
    ],
)

_NB = 2560
_NSTEPS = 20480 // _NB


def _head2_body(ph_ref, pw2_ref, pb2_ref, pol_ref):
    pol_ref[...] = _mm(ph_ref[...], pw2_ref[...]) + pb2_ref[...]


_head2 = pl.pallas_call(
    _head2_body,
    grid=(_NSTEPS,),
    in_specs=[
        pl.BlockSpec((1, 1024), lambda n: (0, 0)),
        pl.BlockSpec((1024, _NB), lambda n: (0, n)),
        pl.BlockSpec((1, _NB), lambda n: (0, n)),
    ],
    out_specs=[pl.BlockSpec((1, _NB), lambda n: (0, n))],
    out_shape=[jax.ShapeDtypeStruct((1, 20480), _F32)],
)


def kernel(node_features, edge_index, edge_attr, params):
    p = params
    src = edge_index[0]
    tgt = edge_index[1]
    convs = (p['conv1'], p['conv2'], p['conv3'])

    x0, ab1, c1 = _prologue(node_features, p['embed_w'],
                            p['embed_b'][None, :], edge_attr,
                            convs[0]['mw1'], convs[0]['mb1'][None, :])
    c2, c3, oht = _prologue2(
        edge_attr, convs[1]['mw1'], convs[2]['mw1'],
        convs[1]['mb1'][None, :], convs[2]['mb1'][None, :], tgt[None, :])

    x, ab = x0, ab1
    cs = (c1, c2, c3)
    for i in range(3):
        c = convs[i]
        m1 = _sc_conv(ab, cs[i], src, tgt)
        if i < 2:
            x, ab = _update(x, m1, oht, c['mw2'], c['mb2'][None, :],
                            c['uw1'], c['ub1'][None, :], c['uw2'],
                            c['ub2'][None, :], convs[i + 1]['mw1'])
        else:
            (x,) = _update_last(x, m1, oht, c['mw2'], c['mb2'][None, :],
                                c['uw1'], c['ub1'][None, :], c['uw2'],
                                c['ub2'][None, :])

    g = x.reshape(1, N * H)
    ph, val = _head1(g, p['pw1'], p['vw1'], p['pb1'][None, :],
                     p['vb1'][None, :], p['vw2'].reshape(1, H),
                     p['vb2'][None, :])
    (pol,) = _head2(ph, p['pw2'], p['pb2'][None, :])
    return pol.reshape(20480), val.reshape(1)
